# baseline (speedup 1.0000x reference)
import jax
import jax.numpy as jnp
from jax import lax
from jax.experimental import pallas as pl
from jax.experimental.pallas import tpu as pltpu
from jax.experimental.pallas import tpu_sc as plsc

_WINDOW = 128
_R = 2048
_TC_FRAC = 0.735


def _tc_lookup(idx_tc, table, n_out):
    m = idx_tc.shape[0]
    vocab, dim = table.shape
    nblk = m // _R
    idx3 = idx_tc.reshape(nblk, 1, _R)

    tpad = jnp.zeros((128, dim), table.dtype).at[:vocab].set(table)
    thi = tpad.astype(jnp.bfloat16)
    tlo = (tpad - thi.astype(jnp.float32)).astype(jnp.bfloat16)

    def body(idx_ref, thi_ref, tlo_ref, o_ref):
        ids = idx_ref[0, 0, :]
        oh = (ids[:, None] == lax.broadcasted_iota(jnp.int32, (_R, 128), 1)).astype(
            jnp.bfloat16
        )
        o_ref[...] = jnp.dot(
            oh, thi_ref[...], preferred_element_type=jnp.float32
        ) + jnp.dot(oh, tlo_ref[...], preferred_element_type=jnp.float32)

    return pl.pallas_call(
        body,
        grid=(nblk,),
        in_specs=[
            pl.BlockSpec((1, 1, _R), lambda i: (i, 0, 0)),
            pl.BlockSpec((128, dim), lambda i: (0, 0)),
            pl.BlockSpec((128, dim), lambda i: (0, 0)),
        ],
        out_specs=pl.BlockSpec((_R, dim), lambda i: (i, 0)),
        out_shape=jax.ShapeDtypeStruct((n_out, dim), table.dtype),
    )(idx3, thi, tlo)


def _sc_lookup(idx_sc, table):
    m = idx_sc.shape[0]
    vocab, dim = table.shape
    np_ = m // 2

    pid = (idx_sc.reshape(np_, 2)[:, 0] * vocab + idx_sc.reshape(np_, 2)[:, 1]).reshape(
        1, np_
    )
    table2 = jnp.concatenate(
        [
            jnp.broadcast_to(table[:, None, :], (vocab, vocab, dim)),
            jnp.broadcast_to(table[None, :, :], (vocab, vocab, dim)),
        ],
        axis=-1,
    ).reshape(vocab * vocab, 2 * dim)

    mesh = plsc.VectorSubcoreMesh(core_axis_name="c", subcore_axis_name="s")

    @pl.kernel(out_type=jax.ShapeDtypeStruct((np_, 2 * dim), table.dtype), mesh=mesh)
    def _gather(table_hbm, idx_hbm, out_hbm):
        def body(i_vmem, o_vmem):
            pltpu.sync_copy(table_hbm.at[i_vmem.at[0]], o_vmem)

        pltpu.emit_pipeline(
            body,
            grid=(np_ // _WINDOW,),
            in_specs=[pl.BlockSpec((1, _WINDOW), index_map=lambda i: (0, i))],
            out_specs=[pl.BlockSpec((_WINDOW, 2 * dim), index_map=lambda i: (i, 0))],
            core_axis_name=("c", "s"),
            dimension_semantics=(pltpu.PARALLEL,),
        )(idx_hbm, out_hbm)

    return _gather(table2, pid).reshape(m, dim)


def kernel(poi_categories, table):
    batch, seq = poi_categories.shape
    vocab, dim = table.shape
    n = batch * seq
    idx = poi_categories.reshape(n).astype(jnp.int32)

    n_tc = int(n * _TC_FRAC) // 8192 * 8192
    n_sc = n - n_tc

    if n_sc == 0:
        out = _tc_lookup(idx, table, n)
    elif n_tc == 0:
        out = _sc_lookup(idx, table)
    else:
        out_tc = _tc_lookup(idx[:n_tc], table, n)
        out_sc = _sc_lookup(idx[n_tc:], table)
        out = lax.dynamic_update_slice(out_tc, out_sc, (n_tc, 0))
    return out.reshape(batch, seq, dim)
